# baseline (speedup 1.0000x reference)
import functools

import jax
import jax.numpy as jnp
from jax import lax
from jax.experimental import pallas as pl
from jax.experimental.pallas import tpu as pltpu
from jax.experimental.pallas import tpu_sc as plsc

L_SEQ = 200
B_PAD = 256
D = 64
N_OUT = 1024
NC = 2
NS = 16
ROWS_PER_W = B_PAD // (NC * NS)


@functools.cache
def _sc_gather():
    mesh = plsc.VectorSubcoreMesh(core_axis_name="c", subcore_axis_name="s")

    @functools.partial(
        pl.kernel,
        mesh=mesh,
        out_type=jax.ShapeDtypeStruct((B_PAD, D), jnp.float32),
        scratch_types=[
            pltpu.VMEM((ROWS_PER_W,), jnp.int32),
            pltpu.VMEM((ROWS_PER_W, D), jnp.float32),
            pltpu.SemaphoreType.DMA,
        ],
        compiler_params=pltpu.CompilerParams(use_tc_tiling_on_sc=False),
    )
    def gather_k(idx_hbm, table_hbm, out_hbm, idx_v, rows_v, sem):
        wid = lax.axis_index("s") * NC + lax.axis_index("c")
        base = wid * ROWS_PER_W
        pltpu.sync_copy(idx_hbm.at[pl.ds(base, ROWS_PER_W)], idx_v)
        pltpu.async_copy(table_hbm.at[idx_v], rows_v, sem).wait()
        pltpu.sync_copy(rows_v, out_hbm.at[pl.ds(base, ROWS_PER_W)])

    return gather_k


def _dense_body(x_ref, w_ref, b_ref, o_ref):
    x = x_ref[...]
    x = jnp.where(x >= 0, x, 0.01 * x)
    y = lax.dot_general(x, w_ref[...], (((1,), (1,)), ((), ())),
                        preferred_element_type=jnp.float32)
    o_ref[...] = jax.nn.sigmoid(y + b_ref[...])


@functools.cache
def _dense_call():
    return pl.pallas_call(
        _dense_body,
        out_shape=jax.ShapeDtypeStruct((B_PAD, N_OUT), jnp.float32),
    )


@jax.jit
def kernel(inputs, emb_table, W, b):
    idx = inputs[0].astype(jnp.int32)
    idx_pad = jnp.zeros((B_PAD,), jnp.int32).at[:L_SEQ].set(idx)
    gathered = _sc_gather()(idx_pad, emb_table)
    out = _dense_call()(gathered, W, b.reshape(1, N_OUT))
    return out[:L_SEQ]


# speedup vs baseline: 44.6769x; 44.6769x over previous
import functools

import jax
import jax.numpy as jnp
from jax import lax
from jax.experimental import pallas as pl
from jax.experimental.pallas import tpu as pltpu

VOCAB_N = 1000000
L_SEQ = 200
D = 64
N_OUT = 1024
BLK = 128
TAIL0 = VOCAB_N // BLK * BLK
MAX_START = TAIL0 - BLK


def _body(idx_smem, idx_vmem, table_ref, tail_ref, wt_ref, b_ref, o_ref,
          blocks, sem):
    for j in range(L_SEQ):
        start = idx_smem[j] // BLK * BLK
        start = pl.multiple_of(jnp.minimum(start, MAX_START), BLK)
        pltpu.make_async_copy(
            table_ref.at[:, pl.ds(start, BLK)], blocks.at[j], sem
        ).start()
    for j in range(L_SEQ):
        pltpu.make_async_copy(
            table_ref.at[:, pl.ds(0, BLK)], blocks.at[j], sem
        ).wait()

    idx_v = idx_vmem[...]
    lane = idx_v - jnp.minimum(idx_v // BLK * BLK, MAX_START)
    lanes_iota = lax.broadcasted_iota(jnp.int32, (L_SEQ, 1, BLK), 2)
    mask = (lanes_iota == lane[:, :, None]).astype(jnp.float32)
    x = jnp.sum(blocks[...] * mask, axis=2)

    tail_lane = idx_v - TAIL0
    tail_mask = (lanes_iota == tail_lane[:, :, None]).astype(jnp.float32)
    x = x + jnp.sum(tail_ref[...][None, :, :] * tail_mask, axis=2)

    x = jnp.where(x >= 0, x, 0.01 * x)
    y = lax.dot_general(x, wt_ref[...], (((1,), (0,)), ((), ())),
                        preferred_element_type=jnp.float32)
    o_ref[...] = jax.nn.sigmoid(y + b_ref[...])


@functools.cache
def _fused_call():
    return pl.pallas_call(
        _body,
        grid=(),
        in_specs=[
            pl.BlockSpec(memory_space=pltpu.SMEM),
            pl.BlockSpec(memory_space=pltpu.VMEM),
            pl.BlockSpec(memory_space=pl.ANY),
            pl.BlockSpec(memory_space=pltpu.VMEM),
            pl.BlockSpec(memory_space=pltpu.VMEM),
            pl.BlockSpec(memory_space=pltpu.VMEM),
        ],
        out_shape=jax.ShapeDtypeStruct((L_SEQ, N_OUT), jnp.float32),
        scratch_shapes=[
            pltpu.VMEM((L_SEQ, D, BLK), jnp.float32),
            pltpu.SemaphoreType.DMA,
        ],
    )


@jax.jit
def kernel(inputs, emb_table, W, b):
    idx = inputs[0].astype(jnp.int32)
    tail = jnp.zeros((D, BLK), jnp.float32)
    tail = tail.at[:, : VOCAB_N - TAIL0].set(emb_table[TAIL0:].T)
    return _fused_call()(
        idx, idx.reshape(L_SEQ, 1), emb_table.T, tail, W.T,
        b.reshape(1, N_OUT)
    )


# speedup vs baseline: 56.1167x; 1.2561x over previous
import functools

import jax
import jax.numpy as jnp
from jax import lax
from jax.experimental import pallas as pl
from jax.experimental.pallas import tpu as pltpu

VOCAB_N = 1000000
L_SEQ = 200
D = 64
N_OUT = 1024
BLK = 128
TAIL0 = VOCAB_N // BLK * BLK
MAX_START = TAIL0 - BLK


NCHUNK = 4
CH = L_SEQ // NCHUNK


def _body(idx_smem, idx_vmem, table_ref, tail_ref, wt_ref, b_ref, o_ref,
          blocks, sems):
    for j in range(L_SEQ):
        start = idx_smem[j] // BLK * BLK
        start = pl.multiple_of(jnp.minimum(start, MAX_START), BLK)
        pltpu.make_async_copy(
            table_ref.at[:, pl.ds(start, BLK)], blocks.at[j],
            sems.at[j // CH],
        ).start()

    idx_v = idx_vmem[...]
    lane = idx_v - jnp.minimum(idx_v // BLK * BLK, MAX_START)
    tail_lane = idx_v - TAIL0
    lanes_iota = lax.broadcasted_iota(jnp.int32, (L_SEQ, 1, BLK), 2)
    cond_m = lanes_iota == lane[:, :, None]
    cond_t = lanes_iota == tail_lane[:, :, None]
    tail_b = tail_ref[...][None, :, :]

    xs = []
    for k in range(NCHUNK):
        for _ in range(CH):
            pltpu.make_async_copy(
                table_ref.at[:, pl.ds(0, BLK)], blocks.at[0], sems.at[k]
            ).wait()
        sl = slice(k * CH, (k + 1) * CH)
        comb = (jnp.where(cond_m[sl], blocks[sl], 0.0)
                + jnp.where(cond_t[sl], tail_b, 0.0))
        xs.append(jnp.sum(comb, axis=2))

    x = jnp.concatenate(xs, axis=0)
    x = jnp.where(x >= 0, x, 0.01 * x)
    y = lax.dot_general(x, wt_ref[...], (((1,), (0,)), ((), ())),
                        preferred_element_type=jnp.float32)
    o_ref[...] = jax.nn.sigmoid(y + b_ref[...])


@functools.cache
def _fused_call():
    return pl.pallas_call(
        _body,
        grid=(),
        in_specs=[
            pl.BlockSpec(memory_space=pltpu.SMEM),
            pl.BlockSpec(memory_space=pltpu.VMEM),
            pl.BlockSpec(memory_space=pl.ANY),
            pl.BlockSpec(memory_space=pltpu.VMEM),
            pl.BlockSpec(memory_space=pltpu.VMEM),
            pl.BlockSpec(memory_space=pltpu.VMEM),
        ],
        out_shape=jax.ShapeDtypeStruct((L_SEQ, N_OUT), jnp.float32),
        scratch_shapes=[
            pltpu.VMEM((L_SEQ, D, BLK), jnp.float32),
            pltpu.SemaphoreType.DMA((NCHUNK,)),
        ],
    )


@jax.jit
def kernel(inputs, emb_table, W, b):
    idx = inputs[0].astype(jnp.int32)
    tail = jnp.zeros((D, BLK), jnp.float32)
    tail = tail.at[:, : VOCAB_N - TAIL0].set(emb_table[TAIL0:].T)
    return _fused_call()(
        idx, idx.reshape(L_SEQ, 1), emb_table.T, tail, W.T,
        b.reshape(1, N_OUT)
    )


# speedup vs baseline: 62.4590x; 1.1130x over previous
import functools

import jax
import jax.numpy as jnp
from jax import lax
from jax.experimental import pallas as pl
from jax.experimental.pallas import tpu as pltpu

VOCAB_N = 1000000
L_SEQ = 200
D = 64
N_OUT = 1024
BLK = 128
TAIL0 = VOCAB_N // BLK * BLK
TAIL_W = VOCAB_N - TAIL0
MAX_START = TAIL0 - BLK
NCHUNK = 8
CH = L_SEQ // NCHUNK


def _body(idx_smem, idx_vmem, table_ref, wt_ref, b_ref, o_ref,
          blocks, tailbuf, sems, tail_sem):
    pltpu.make_async_copy(
        table_ref.at[:, pl.ds(TAIL0, TAIL_W)], tailbuf, tail_sem
    ).start()

    for j in range(L_SEQ):
        start = idx_smem[j] // BLK * BLK
        start = pl.multiple_of(jnp.minimum(start, MAX_START), BLK)
        pltpu.make_async_copy(
            table_ref.at[:, pl.ds(start, BLK)], blocks.at[j],
            sems.at[j // CH],
        ).start()

    idx_v = idx_vmem[...]
    lane = idx_v - jnp.minimum(idx_v // BLK * BLK, MAX_START)
    lanes_iota = lax.broadcasted_iota(jnp.int32, (L_SEQ, 1, BLK), 2)
    cond_m = lanes_iota == lane[:, :, None]
    cond_t = lanes_iota == (idx_v - TAIL0)[:, :, None]

    pltpu.make_async_copy(
        table_ref.at[:, pl.ds(TAIL0, TAIL_W)], tailbuf, tail_sem
    ).wait()
    tail_b = jnp.concatenate(
        [tailbuf[...], jnp.zeros((D, BLK - TAIL_W), jnp.float32)], axis=1
    )[None]

    xs = []
    for k in range(NCHUNK):
        for _ in range(CH):
            pltpu.make_async_copy(
                table_ref.at[:, pl.ds(0, BLK)], blocks.at[0], sems.at[k]
            ).wait()
        sl = slice(k * CH, (k + 1) * CH)
        comb = (jnp.where(cond_m[sl], blocks[sl], 0.0)
                + jnp.where(cond_t[sl], tail_b, 0.0))
        xs.append(jnp.sum(comb, axis=2))

    x = jnp.concatenate(xs, axis=0)
    x = jnp.where(x >= 0, x, 0.01 * x)
    y = lax.dot_general(x, wt_ref[...], (((1,), (0,)), ((), ())),
                        preferred_element_type=jnp.float32)
    o_ref[...] = jax.nn.sigmoid(y + b_ref[...])


@functools.cache
def _fused_call():
    return pl.pallas_call(
        _body,
        grid=(),
        in_specs=[
            pl.BlockSpec(memory_space=pltpu.SMEM),
            pl.BlockSpec(memory_space=pltpu.VMEM),
            pl.BlockSpec(memory_space=pl.ANY),
            pl.BlockSpec(memory_space=pltpu.VMEM),
            pl.BlockSpec(memory_space=pltpu.VMEM),
        ],
        out_shape=jax.ShapeDtypeStruct((L_SEQ, N_OUT), jnp.float32),
        scratch_shapes=[
            pltpu.VMEM((L_SEQ, D, BLK), jnp.float32),
            pltpu.VMEM((D, TAIL_W), jnp.float32),
            pltpu.SemaphoreType.DMA((NCHUNK,)),
            pltpu.SemaphoreType.DMA,
        ],
    )


@jax.jit
def kernel(inputs, emb_table, W, b):
    idx = inputs[0].astype(jnp.int32)
    return _fused_call()(
        idx, idx.reshape(L_SEQ, 1), emb_table.T, W.T, b.reshape(1, N_OUT)
    )


# speedup vs baseline: 70.2103x; 1.1241x over previous
import functools

import jax
import jax.numpy as jnp
from jax import lax
from jax.experimental import pallas as pl
from jax.experimental.pallas import tpu as pltpu

VOCAB_N = 1000000
L_SEQ = 200
D = 64
N_OUT = 1024
BLK = 128
TAIL0 = VOCAB_N // BLK * BLK
TAIL_W = VOCAB_N - TAIL0
MAX_START = TAIL0 - BLK
NCHUNK = 8
CH = L_SEQ // NCHUNK


def _body(idx_smem, idx_vmem, table_ref, wt_ref, b_ref, o_ref,
          blocks, tailbuf, sems, tail_sem):
    pltpu.make_async_copy(
        table_ref.at[:, pl.ds(TAIL0, TAIL_W)], tailbuf, tail_sem
    ).start()

    for j in range(L_SEQ):
        start = idx_smem[j] // BLK * BLK
        start = pl.multiple_of(jnp.minimum(start, MAX_START), BLK)
        pltpu.make_async_copy(
            table_ref.at[:, pl.ds(start, BLK)], blocks.at[j],
            sems.at[j // CH],
        ).start()

    idx_v = idx_vmem[...]
    lane = idx_v - jnp.minimum(idx_v // BLK * BLK, MAX_START)
    lanes_iota = lax.broadcasted_iota(jnp.int32, (L_SEQ, 1, BLK), 2)
    cond_m = lanes_iota == lane[:, :, None]
    cond_t = (lanes_iota[:, 0, :] == idx_v - TAIL0).astype(jnp.float32)

    xs = []
    for k in range(NCHUNK):
        for _ in range(CH):
            pltpu.make_async_copy(
                table_ref.at[:, pl.ds(0, BLK)], blocks.at[0], sems.at[k]
            ).wait()
        sl = slice(k * CH, (k + 1) * CH)
        xs.append(jnp.sum(jnp.where(cond_m[sl], blocks[sl], 0.0), axis=2))

    x = jnp.concatenate(xs, axis=0)

    pltpu.make_async_copy(
        table_ref.at[:, pl.ds(TAIL0, TAIL_W)], tailbuf, tail_sem
    ).wait()
    tail_mat = jnp.concatenate(
        [tailbuf[...].T, jnp.zeros((BLK - TAIL_W, D), jnp.float32)], axis=0
    )
    x = x + lax.dot_general(cond_t, tail_mat, (((1,), (0,)), ((), ())),
                            preferred_element_type=jnp.float32)
    x = jnp.where(x >= 0, x, 0.01 * x)
    y = lax.dot_general(x, wt_ref[...], (((1,), (0,)), ((), ())),
                        preferred_element_type=jnp.float32)
    o_ref[...] = jax.nn.sigmoid(y + b_ref[...])


@functools.cache
def _fused_call():
    return pl.pallas_call(
        _body,
        grid=(),
        in_specs=[
            pl.BlockSpec(memory_space=pltpu.SMEM),
            pl.BlockSpec(memory_space=pltpu.VMEM),
            pl.BlockSpec(memory_space=pl.ANY),
            pl.BlockSpec(memory_space=pltpu.VMEM),
            pl.BlockSpec(memory_space=pltpu.VMEM),
        ],
        out_shape=jax.ShapeDtypeStruct((L_SEQ, N_OUT), jnp.float32),
        scratch_shapes=[
            pltpu.VMEM((L_SEQ, D, BLK), jnp.float32),
            pltpu.VMEM((D, TAIL_W), jnp.float32),
            pltpu.SemaphoreType.DMA((NCHUNK,)),
            pltpu.SemaphoreType.DMA,
        ],
    )


@jax.jit
def kernel(inputs, emb_table, W, b):
    idx = inputs[0].astype(jnp.int32)
    return _fused_call()(
        idx, idx.reshape(L_SEQ, 1), emb_table.T, W.T, b.reshape(1, N_OUT)
    )
